# Initial kernel scaffold; baseline (speedup 1.0000x reference)
import jax
import jax.numpy as jnp
from jax.experimental import pallas as pl


def kernel(edge_attr, edge_index, W, att_l, att_r, W_res):
    raise NotImplementedError("write your pallas kernel here")


# speedup vs baseline: 27.4413x; 27.4413x over previous
import functools

import jax
import jax.numpy as jnp
from jax import lax
from jax.experimental import pallas as pl
from jax.experimental.pallas import tpu as pltpu
from jax.experimental.pallas import tpu_sc as plsc

N_NODES = 10000
E_RAND = 320000
E_TOTAL = E_RAND + N_NODES
D = 128
H = 4
C = 32

NC = 2
NS = 16
L = 16

G = 128
GROUPS = 81
E_PAD = NC * NS * GROUPS * G
N_ACC = 10112
ROWS = N_ACC // NS

EB = 512
NB = 1000


def _hsel():
    hh = lax.broadcasted_iota(jnp.int32, (H, H * C), 0)
    jj = lax.broadcasted_iota(jnp.int32, (H, H * C), 1)
    return (jj // C == hh).astype(jnp.float32)


def _alpha_l_body(ea_ref, w_ref, attl_ref, out_ref):
    wv = w_ref[:, :] * attl_ref[0, :][:, None]
    vlt = lax.dot_general(_hsel(), wv, (((1,), (0,)), ((), ())),
                          preferred_element_type=jnp.float32)
    out_ref[:, :] = lax.dot_general(vlt, ea_ref[:, :], (((1,), (1,)), ((), ())),
                                    preferred_element_type=jnp.float32)


def _node_body(f_ref, w_ref, attr_ref, wres_ref, nr_ref, res_ref):
    f = f_ref[:, :]
    xi = lax.dot_general(f, w_ref[:, :], (((1,), (1,)), ((), ())),
                         preferred_element_type=jnp.float32)
    xia = xi * attr_ref[0, :][None, :]
    nr_ref[:, :] = lax.dot_general(_hsel(), xia, (((1,), (1,)), ((), ())),
                                   preferred_element_type=jnp.float32)
    res_ref[:, :] = lax.dot_general(f, wres_ref[:, :], (((1,), (1,)), ((), ())),
                                    preferred_element_type=jnp.float32)


def _y_body(ea_ref, ex_ref, w_ref, y_ref):
    xj = lax.dot_general(ea_ref[:, :], w_ref[:, :], (((1,), (1,)), ((), ())),
                         preferred_element_type=jnp.float32)
    scale = lax.dot_general(ex_ref[:, :], _hsel(), (((0,), (0,)), ((), ())),
                            preferred_element_type=jnp.float32)
    y_ref[:, :] = xj * scale


def _final_body(sp_ref, dp_ref, res_ref, out_ref):
    s = sp_ref[0] + sp_ref[1]
    d = dp_ref[0] + dp_ref[1]
    dexp = lax.dot_general(d, _hsel(), (((1,), (0,)), ((), ())),
                           preferred_element_type=jnp.float32) + 1e-16
    pre = s / dexp
    act = jnp.where(pre > 0, pre, jnp.exp(jnp.minimum(pre, 0.0)) - 1.0)
    out_ref[:, :] = act + res_ref[:, :]


_SC_MESH = plsc.VectorSubcoreMesh(core_axis_name="c", subcore_axis_name="s")


@functools.partial(
    pl.kernel,
    out_type=[
        jax.ShapeDtypeStruct((H * E_PAD,), jnp.float32),
        jax.ShapeDtypeStruct((NC * N_ACC * H,), jnp.float32),
    ],
    mesh=_SC_MESH,
    scratch_types=[
        pltpu.VMEM((H * N_NODES,), jnp.float32),
        pltpu.VMEM((H * G,), jnp.float32),
        pltpu.VMEM((H * G,), jnp.float32),
        pltpu.VMEM((G,), jnp.int32),
        pltpu.VMEM((G,), jnp.int32),
        pltpu.VMEM((G,), jnp.int32),
        pltpu.VMEM((G,), jnp.int32),
        pltpu.VMEM((G,), jnp.int32),
        pltpu.VMEM((G,), jnp.int32),
        pltpu.VMEM_SHARED((N_ACC * H,), jnp.float32),
    ],
    compiler_params=pltpu.CompilerParams(use_tc_tiling_on_sc=False, needs_layout_passes=False),
)
def _sc_softmax(al_hbm, nr_hbm, tgt_hbm, src_hbm, zden_hbm,
                ex_hbm, dparts_hbm,
                nr_v, al_v, ex_v, tgt_v, src_v, i0_v, i1_v, i2_v, i3_v, dacc):
    c = lax.axis_index("c")
    s = lax.axis_index("s")
    wid = c * NS + s
    idx_refs = (i0_v, i1_v, i2_v, i3_v)

    pltpu.sync_copy(zden_hbm.at[pl.ds(s * ROWS * H, ROWS * H)],
                    dacc.at[pl.ds(s * ROWS * H, ROWS * H)])
    pltpu.sync_copy(nr_hbm, nr_v)
    plsc.subcore_barrier()

    @pl.loop(0, GROUPS)
    def _group(g):
        base = (wid * GROUPS + g) * G
        for h in range(H):
            pltpu.sync_copy(al_hbm.at[pl.ds(h * E_PAD + base, G)],
                            al_v.at[pl.ds(h * G, G)])
        pltpu.sync_copy(tgt_hbm.at[pl.ds(base, G)], tgt_v)
        pltpu.sync_copy(src_hbm.at[pl.ds(base, G)], src_v)
        for h in range(H):
            for j in range(G // L):
                tg = tgt_v[pl.ds(j * L, L)]
                r = plsc.load_gather(nr_v, [tg + h * N_NODES])
                a = al_v[pl.ds(h * G + j * L, L)] + r
                ex = jnp.exp(jnp.maximum(a, 0.2 * a))
                ex_v[pl.ds(h * G + j * L, L)] = ex
                sv = src_v[pl.ds(j * L, L)]
                idx_refs[h][pl.ds(j * L, L)] = sv * H + h
        for h in range(H):
            pltpu.sync_copy(ex_v.at[pl.ds(h * G, G)],
                            ex_hbm.at[pl.ds(h * E_PAD + base, G)])
        for h in range(H):
            pltpu.sync_copy(ex_v.at[pl.ds(h * G, G)],
                            dacc.at[idx_refs[h]], add=True)

    plsc.subcore_barrier()
    pltpu.sync_copy(dacc.at[pl.ds(s * ROWS * H, ROWS * H)],
                    dparts_hbm.at[pl.ds((c * NS + s) * ROWS * H, ROWS * H)])


@functools.partial(
    pl.kernel,
    out_type=jax.ShapeDtypeStruct((NC, N_ACC, D), jnp.float32),
    mesh=_SC_MESH,
    scratch_types=[
        pltpu.VMEM((G, D), jnp.float32),
        pltpu.VMEM((G,), jnp.int32),
        pltpu.VMEM_SHARED((N_ACC, D), jnp.float32),
    ],
    compiler_params=pltpu.CompilerParams(use_tc_tiling_on_sc=False, needs_layout_passes=False),
)
def _sc_scatter(y_hbm, src_hbm, zout_hbm, oparts_hbm, y_v, src_v, oacc):
    c = lax.axis_index("c")
    s = lax.axis_index("s")
    wid = c * NS + s

    pltpu.sync_copy(zout_hbm.at[pl.ds(s * ROWS, ROWS)], oacc.at[pl.ds(s * ROWS, ROWS)])
    plsc.subcore_barrier()

    @pl.loop(0, GROUPS)
    def _group(g):
        base = (wid * GROUPS + g) * G
        pltpu.sync_copy(y_hbm.at[pl.ds(base, G)], y_v)
        pltpu.sync_copy(src_hbm.at[pl.ds(base, G)], src_v)
        pltpu.sync_copy(y_v, oacc.at[src_v], add=True)

    plsc.subcore_barrier()
    pltpu.sync_copy(oacc.at[pl.ds(s * ROWS, ROWS)],
                    oparts_hbm.at[c, pl.ds(s * ROWS, ROWS)])


def kernel(edge_attr, edge_index, W, att_l, att_r, W_res):
    src = edge_index[1].astype(jnp.int32)
    tgt = edge_index[0].astype(jnp.int32)
    npad = E_PAD - E_TOTAL
    pad_src = (jnp.arange(npad, dtype=jnp.int32) % L) + N_NODES
    src_p = jnp.concatenate([src, pad_src])
    tgt_p = jnp.concatenate([tgt, jnp.zeros((npad,), jnp.int32)])

    filtered = edge_attr[E_RAND:]
    attl2 = att_l.reshape(1, H * C)
    attr2 = att_r.reshape(1, H * C)
    zden = jnp.zeros((N_ACC * H,), jnp.float32)
    zout = jnp.zeros((N_ACC, D), jnp.float32)

    egrid = pl.cdiv(E_TOTAL, EB)

    alpha_l = pl.pallas_call(
        _alpha_l_body,
        grid=(egrid,),
        in_specs=[
            pl.BlockSpec((EB, D), lambda i: (i, 0)),
            pl.BlockSpec((H * C, D), lambda i: (0, 0)),
            pl.BlockSpec((1, H * C), lambda i: (0, 0)),
        ],
        out_specs=pl.BlockSpec((H, EB), lambda i: (0, i)),
        out_shape=jax.ShapeDtypeStruct((H, E_PAD), jnp.float32),
    )(edge_attr, W, attl2)

    node_r, res = pl.pallas_call(
        _node_body,
        grid=(pl.cdiv(N_NODES, 128),),
        in_specs=[
            pl.BlockSpec((128, D), lambda i: (i, 0)),
            pl.BlockSpec((H * C, D), lambda i: (0, 0)),
            pl.BlockSpec((1, H * C), lambda i: (0, 0)),
            pl.BlockSpec((H * C, D), lambda i: (0, 0)),
        ],
        out_specs=[
            pl.BlockSpec((H, 128), lambda i: (0, i)),
            pl.BlockSpec((128, H * C), lambda i: (i, 0)),
        ],
        out_shape=[
            jax.ShapeDtypeStruct((H, N_NODES), jnp.float32),
            jax.ShapeDtypeStruct((N_NODES, H * C), jnp.float32),
        ],
    )(filtered, W, attr2, W_res)

    ex_flat, dparts_flat = _sc_softmax(alpha_l.reshape(-1), node_r.reshape(-1),
                                       tgt_p, src_p, zden)
    ex = ex_flat.reshape(H, E_PAD)
    dparts = dparts_flat.reshape(NC, N_ACC, H)

    y = pl.pallas_call(
        _y_body,
        grid=(egrid,),
        in_specs=[
            pl.BlockSpec((EB, D), lambda i: (i, 0)),
            pl.BlockSpec((H, EB), lambda i: (0, i)),
            pl.BlockSpec((H * C, D), lambda i: (0, 0)),
        ],
        out_specs=pl.BlockSpec((EB, D), lambda i: (i, 0)),
        out_shape=jax.ShapeDtypeStruct((E_PAD, D), jnp.float32),
    )(edge_attr, ex, W)

    oparts = _sc_scatter(y, src_p, zout)

    out = pl.pallas_call(
        _final_body,
        grid=(N_NODES // NB,),
        in_specs=[
            pl.BlockSpec((NC, NB, D), lambda i: (0, i, 0)),
            pl.BlockSpec((NC, NB, H), lambda i: (0, i, 0)),
            pl.BlockSpec((NB, H * C), lambda i: (i, 0)),
        ],
        out_specs=pl.BlockSpec((NB, H * C), lambda i: (i, 0)),
        out_shape=jax.ShapeDtypeStruct((N_NODES, H * C), jnp.float32),
    )(oparts, dparts, res)

    return out


# speedup vs baseline: 35.0977x; 1.2790x over previous
import functools

import jax
import jax.numpy as jnp
from jax import lax
from jax.experimental import pallas as pl
from jax.experimental.pallas import tpu as pltpu
from jax.experimental.pallas import tpu_sc as plsc

N_NODES = 10000
E_RAND = 320000
E_TOTAL = E_RAND + N_NODES
D = 128
H = 4
C = 32

NC = 2
NS = 16
L = 16

G = 128
GROUPS = 81
E_PAD = NC * NS * GROUPS * G
N_ACC = 10112
ROWS = N_ACC // NS

EB = 512
NB = 1000


def _hsel():
    hh = lax.broadcasted_iota(jnp.int32, (H, H * C), 0)
    jj = lax.broadcasted_iota(jnp.int32, (H, H * C), 1)
    return (jj // C == hh).astype(jnp.float32)


def _alpha_l_body(ea_ref, w_ref, attl_ref, out_ref):
    wv = w_ref[:, :] * attl_ref[0, :][:, None]
    vlt = lax.dot_general(_hsel(), wv, (((1,), (0,)), ((), ())),
                          preferred_element_type=jnp.float32)
    out_ref[:, :] = lax.dot_general(vlt, ea_ref[:, :], (((1,), (1,)), ((), ())),
                                    preferred_element_type=jnp.float32)


def _node_body(f_ref, w_ref, attr_ref, wres_ref, nr_ref, res_ref):
    f = f_ref[:, :]
    xi = lax.dot_general(f, w_ref[:, :], (((1,), (1,)), ((), ())),
                         preferred_element_type=jnp.float32)
    xia = xi * attr_ref[0, :][None, :]
    nr_ref[:, :] = lax.dot_general(_hsel(), xia, (((1,), (1,)), ((), ())),
                                   preferred_element_type=jnp.float32)
    res_ref[:, :] = lax.dot_general(f, wres_ref[:, :], (((1,), (1,)), ((), ())),
                                    preferred_element_type=jnp.float32)


def _y_body(ea_ref, ex_ref, w_ref, y_ref):
    xj = lax.dot_general(ea_ref[:, :], w_ref[:, :], (((1,), (1,)), ((), ())),
                         preferred_element_type=jnp.float32)
    scale = lax.dot_general(ex_ref[:, :], _hsel(), (((0,), (0,)), ((), ())),
                            preferred_element_type=jnp.float32)
    y_ref[:, :] = xj * scale


def _final_body(sp_ref, dp_ref, res_ref, out_ref):
    s = sp_ref[0] + sp_ref[1]
    d = dp_ref[0] + dp_ref[1]
    dexp = lax.dot_general(d, _hsel(), (((1,), (0,)), ((), ())),
                           preferred_element_type=jnp.float32) + 1e-16
    pre = s / dexp
    act = jnp.where(pre > 0, pre, jnp.exp(jnp.minimum(pre, 0.0)) - 1.0)
    out_ref[:, :] = act + res_ref[:, :]


_SC_MESH = plsc.VectorSubcoreMesh(core_axis_name="c", subcore_axis_name="s")


SUBG = 27
NSUB = GROUPS // SUBG
SE = SUBG * G


@functools.partial(
    pl.kernel,
    out_type=[
        jax.ShapeDtypeStruct((H * E_PAD,), jnp.float32),
        jax.ShapeDtypeStruct((NC * N_ACC * H,), jnp.float32),
    ],
    mesh=_SC_MESH,
    scratch_types=[
        pltpu.VMEM((H * N_NODES,), jnp.float32),
        pltpu.VMEM((H * SE,), jnp.float32),
        pltpu.VMEM((H * SE,), jnp.float32),
        pltpu.VMEM((SE,), jnp.int32),
        pltpu.VMEM((SE,), jnp.int32),
        pltpu.VMEM((H * SUBG, G), jnp.int32),
        pltpu.VMEM_SHARED((N_ACC * H,), jnp.float32),
        pltpu.SemaphoreType.DMA,
    ],
    compiler_params=pltpu.CompilerParams(use_tc_tiling_on_sc=False, needs_layout_passes=False),
)
def _sc_softmax(al_hbm, nr_hbm, tgt_hbm, src_hbm, zden_hbm,
                ex_hbm, dparts_hbm,
                nr_v, al_v, ex_v, tgt_v, src_v, idx_v, dacc, ssem):
    c = lax.axis_index("c")
    s = lax.axis_index("s")
    wid = c * NS + s

    pltpu.sync_copy(zden_hbm.at[pl.ds(s * ROWS * H, ROWS * H)],
                    dacc.at[pl.ds(s * ROWS * H, ROWS * H)])
    pltpu.sync_copy(nr_hbm, nr_v)
    plsc.subcore_barrier()

    @pl.loop(0, NSUB)
    def _sub(sub):
        base = wid * GROUPS * G + sub * SE
        for h in range(H):
            pltpu.sync_copy(al_hbm.at[pl.ds(h * E_PAD + base, SE)],
                            al_v.at[pl.ds(h * SE, SE)])
        pltpu.sync_copy(tgt_hbm.at[pl.ds(base, SE)], tgt_v)
        pltpu.sync_copy(src_hbm.at[pl.ds(base, SE)], src_v)

        @pl.loop(0, SUBG)
        def _group(gg):
            for h in range(H):
                for j in range(G // L):
                    o = gg * G + j * L
                    tg = tgt_v[pl.ds(o, L)]
                    r = plsc.load_gather(nr_v, [tg + h * N_NODES])
                    a = al_v[pl.ds(h * SE + o, L)] + r
                    ex = jnp.exp(jnp.maximum(a, 0.2 * a))
                    ex_v[pl.ds(h * SE + o, L)] = ex
                    sv = src_v[pl.ds(o, L)]
                    idx_v[h * SUBG + gg, pl.ds(j * L, L)] = sv * H + h

        for h in range(H):
            pltpu.sync_copy(ex_v.at[pl.ds(h * SE, SE)],
                            ex_hbm.at[pl.ds(h * E_PAD + base, SE)])

        @pl.loop(0, SUBG)
        def _fire(gg):
            for h in range(H):
                pltpu.async_copy(ex_v.at[pl.ds(h * SE + gg * G, G)],
                                 dacc.at[idx_v.at[h * SUBG + gg]],
                                 ssem, add=True)

        pltpu.make_async_copy(al_hbm.at[pl.ds(0, H * SE)], ex_v, ssem).wait()

    plsc.subcore_barrier()
    pltpu.sync_copy(dacc.at[pl.ds(s * ROWS * H, ROWS * H)],
                    dparts_hbm.at[pl.ds((c * NS + s) * ROWS * H, ROWS * H)])


@functools.partial(
    pl.kernel,
    out_type=jax.ShapeDtypeStruct((NC, N_ACC, D), jnp.float32),
    mesh=_SC_MESH,
    scratch_types=[
        pltpu.VMEM((G, D), jnp.float32),
        pltpu.VMEM((G, D), jnp.float32),
        pltpu.VMEM((GROUPS, G), jnp.int32),
        pltpu.VMEM_SHARED((N_ACC, D), jnp.float32),
        pltpu.SemaphoreType.DMA,
        pltpu.SemaphoreType.DMA,
        pltpu.SemaphoreType.DMA,
        pltpu.SemaphoreType.DMA,
    ],
    compiler_params=pltpu.CompilerParams(use_tc_tiling_on_sc=False, needs_layout_passes=False),
)
def _sc_scatter(y_hbm, src_hbm, zout_hbm, oparts_hbm,
                y0, y1, srcall, oacc, l0, l1, s0, s1):
    c = lax.axis_index("c")
    s = lax.axis_index("s")
    wid = c * NS + s
    ybufs = (y0, y1)
    lsems = (l0, l1)
    ssems = (s0, s1)

    pltpu.sync_copy(zout_hbm.at[pl.ds(s * ROWS, ROWS)], oacc.at[pl.ds(s * ROWS, ROWS)])
    pltpu.sync_copy(src_hbm.at[pl.ds(wid * GROUPS, GROUPS)], srcall)
    plsc.subcore_barrier()

    tbase = wid * GROUPS * G
    pltpu.async_copy(y_hbm.at[pl.ds(tbase, G)], y0, l0)
    pltpu.async_copy(y_hbm.at[pl.ds(tbase + G, G)], y1, l1)

    @pl.loop(0, GROUPS, step=2)
    def _g(g0):
        for b in range(2):
            g = g0 + b
            bo = 1 - b

            @pl.when(g < GROUPS)
            def _():
                pltpu.make_async_copy(y_hbm.at[pl.ds(0, G)], ybufs[b], lsems[b]).wait()
                pltpu.async_copy(ybufs[b], oacc.at[srcall.at[g]], ssems[b], add=True)

                @pl.when(jnp.logical_and(g >= 1, g + 1 < GROUPS))
                def _():
                    pltpu.make_async_copy(y_hbm.at[pl.ds(0, G)], ybufs[bo], ssems[bo]).wait()
                    pltpu.async_copy(y_hbm.at[pl.ds(tbase + (g + 1) * G, G)],
                                     ybufs[bo], lsems[bo])

    pltpu.make_async_copy(y_hbm.at[pl.ds(0, G)], y0, s0).wait()
    pltpu.make_async_copy(y_hbm.at[pl.ds(0, G)], y1, s1).wait()

    plsc.subcore_barrier()
    pltpu.sync_copy(oacc.at[pl.ds(s * ROWS, ROWS)],
                    oparts_hbm.at[c, pl.ds(s * ROWS, ROWS)])


def kernel(edge_attr, edge_index, W, att_l, att_r, W_res):
    src = edge_index[1].astype(jnp.int32)
    tgt = edge_index[0].astype(jnp.int32)
    npad = E_PAD - E_TOTAL
    pad_src = (jnp.arange(npad, dtype=jnp.int32) % L) + N_NODES
    src_p = jnp.concatenate([src, pad_src])
    tgt_p = jnp.concatenate([tgt, jnp.zeros((npad,), jnp.int32)])

    filtered = edge_attr[E_RAND:]
    attl2 = att_l.reshape(1, H * C)
    attr2 = att_r.reshape(1, H * C)
    zden = jnp.zeros((N_ACC * H,), jnp.float32)
    zout = jnp.zeros((N_ACC, D), jnp.float32)

    egrid = pl.cdiv(E_TOTAL, EB)

    alpha_l = pl.pallas_call(
        _alpha_l_body,
        grid=(egrid,),
        in_specs=[
            pl.BlockSpec((EB, D), lambda i: (i, 0)),
            pl.BlockSpec((H * C, D), lambda i: (0, 0)),
            pl.BlockSpec((1, H * C), lambda i: (0, 0)),
        ],
        out_specs=pl.BlockSpec((H, EB), lambda i: (0, i)),
        out_shape=jax.ShapeDtypeStruct((H, E_PAD), jnp.float32),
    )(edge_attr, W, attl2)

    node_r, res = pl.pallas_call(
        _node_body,
        grid=(pl.cdiv(N_NODES, 128),),
        in_specs=[
            pl.BlockSpec((128, D), lambda i: (i, 0)),
            pl.BlockSpec((H * C, D), lambda i: (0, 0)),
            pl.BlockSpec((1, H * C), lambda i: (0, 0)),
            pl.BlockSpec((H * C, D), lambda i: (0, 0)),
        ],
        out_specs=[
            pl.BlockSpec((H, 128), lambda i: (0, i)),
            pl.BlockSpec((128, H * C), lambda i: (i, 0)),
        ],
        out_shape=[
            jax.ShapeDtypeStruct((H, N_NODES), jnp.float32),
            jax.ShapeDtypeStruct((N_NODES, H * C), jnp.float32),
        ],
    )(filtered, W, attr2, W_res)

    ex_flat, dparts_flat = _sc_softmax(alpha_l.reshape(-1), node_r.reshape(-1),
                                       tgt_p, src_p, zden)
    ex = ex_flat.reshape(H, E_PAD)
    dparts = dparts_flat.reshape(NC, N_ACC, H)

    y = pl.pallas_call(
        _y_body,
        grid=(egrid,),
        in_specs=[
            pl.BlockSpec((EB, D), lambda i: (i, 0)),
            pl.BlockSpec((H, EB), lambda i: (0, i)),
            pl.BlockSpec((H * C, D), lambda i: (0, 0)),
        ],
        out_specs=pl.BlockSpec((EB, D), lambda i: (i, 0)),
        out_shape=jax.ShapeDtypeStruct((E_PAD, D), jnp.float32),
    )(edge_attr, ex, W)

    oparts = _sc_scatter(y, src_p.reshape(NC * NS * GROUPS, G), zout)

    out = pl.pallas_call(
        _final_body,
        grid=(N_NODES // NB,),
        in_specs=[
            pl.BlockSpec((NC, NB, D), lambda i: (0, i, 0)),
            pl.BlockSpec((NC, NB, H), lambda i: (0, i, 0)),
            pl.BlockSpec((NB, H * C), lambda i: (i, 0)),
        ],
        out_specs=pl.BlockSpec((NB, H * C), lambda i: (i, 0)),
        out_shape=jax.ShapeDtypeStruct((N_NODES, H * C), jnp.float32),
    )(oparts, dparts, res)

    return out


# speedup vs baseline: 82.9018x; 2.3620x over previous
import functools

import jax
import jax.numpy as jnp
from jax import lax
from jax.experimental import pallas as pl
from jax.experimental.pallas import tpu as pltpu
from jax.experimental.pallas import tpu_sc as plsc

N_NODES = 10000
E_RAND = 320000
E_TOTAL = E_RAND + N_NODES
D = 128
H = 4
C = 32

NC = 2
NS = 16
L = 16

G = 128
GROUPS = 81
E_PAD = NC * NS * GROUPS * G
N_ACC = 10112
ROWS = N_ACC // NS

EB = 4096
NB = 1000


def _hsel():
    hh = lax.broadcasted_iota(jnp.int32, (H, H * C), 0)
    jj = lax.broadcasted_iota(jnp.int32, (H, H * C), 1)
    return (jj // C == hh).astype(jnp.float32)


def _alpha_l_body(ea_ref, w_ref, attl_ref, out_ref):
    wv = w_ref[:, :] * attl_ref[0, :][:, None]
    vlt = lax.dot_general(_hsel(), wv, (((1,), (0,)), ((), ())),
                          preferred_element_type=jnp.float32)
    out_ref[:, :] = lax.dot_general(vlt, ea_ref[:, :], (((1,), (1,)), ((), ())),
                                    preferred_element_type=jnp.float32)


def _node_body(f_ref, w_ref, attr_ref, wres_ref, nr_ref, res_ref):
    f = f_ref[:, :]
    xi = lax.dot_general(f, w_ref[:, :], (((1,), (1,)), ((), ())),
                         preferred_element_type=jnp.float32)
    xia = xi * attr_ref[0, :][None, :]
    nr_ref[:, :] = lax.dot_general(_hsel(), xia, (((1,), (1,)), ((), ())),
                                   preferred_element_type=jnp.float32)
    res_ref[:, :] = lax.dot_general(f, wres_ref[:, :], (((1,), (1,)), ((), ())),
                                    preferred_element_type=jnp.float32)


def _y_body(ea_ref, ex_ref, w_ref, y_ref):
    xj = lax.dot_general(ea_ref[:, :], w_ref[:, :], (((1,), (1,)), ((), ())),
                         preferred_element_type=jnp.float32)
    scale = lax.dot_general(ex_ref[:, :], _hsel(), (((0,), (0,)), ((), ())),
                            preferred_element_type=jnp.float32)
    y_ref[:, :] = xj * scale


def _final_body(sp_ref, dp_ref, res_ref, out_ref):
    s = sp_ref[0] + sp_ref[1]
    d = dp_ref[0] + dp_ref[1]
    dexp = lax.dot_general(d, _hsel(), (((1,), (0,)), ((), ())),
                           preferred_element_type=jnp.float32) + 1e-16
    pre = s / dexp
    act = jnp.where(pre > 0, pre, jnp.exp(jnp.minimum(pre, 0.0)) - 1.0)
    out_ref[:, :] = act + res_ref[:, :]


_SC_MESH = plsc.VectorSubcoreMesh(core_axis_name="c", subcore_axis_name="s")


SUBG = 27
NSUB = GROUPS // SUBG
SE = SUBG * G


@functools.partial(
    pl.kernel,
    out_type=[
        jax.ShapeDtypeStruct((H * E_PAD,), jnp.float32),
        jax.ShapeDtypeStruct((NC * N_ACC * H,), jnp.float32),
    ],
    mesh=_SC_MESH,
    scratch_types=[
        pltpu.VMEM((H * N_NODES,), jnp.float32),
        pltpu.VMEM((H * SE,), jnp.float32),
        pltpu.VMEM((H * SE,), jnp.float32),
        pltpu.VMEM((SE,), jnp.int32),
        pltpu.VMEM((SE,), jnp.int32),
        pltpu.VMEM((H * SUBG, G), jnp.int32),
        pltpu.VMEM_SHARED((N_ACC * H,), jnp.float32),
        pltpu.SemaphoreType.DMA,
    ],
    compiler_params=pltpu.CompilerParams(use_tc_tiling_on_sc=False, needs_layout_passes=False),
)
def _sc_softmax(al_hbm, nr_hbm, tgt_hbm, src_hbm, zden_hbm,
                ex_hbm, dparts_hbm,
                nr_v, al_v, ex_v, tgt_v, src_v, idx_v, dacc, ssem):
    c = lax.axis_index("c")
    s = lax.axis_index("s")
    wid = c * NS + s

    pltpu.sync_copy(zden_hbm.at[pl.ds(s * ROWS * H, ROWS * H)],
                    dacc.at[pl.ds(s * ROWS * H, ROWS * H)])
    pltpu.sync_copy(nr_hbm, nr_v)
    plsc.subcore_barrier()

    @pl.loop(0, NSUB)
    def _sub(sub):
        base = wid * GROUPS * G + sub * SE
        for h in range(H):
            pltpu.sync_copy(al_hbm.at[pl.ds(h * E_PAD + base, SE)],
                            al_v.at[pl.ds(h * SE, SE)])
        pltpu.sync_copy(tgt_hbm.at[pl.ds(base, SE)], tgt_v)
        pltpu.sync_copy(src_hbm.at[pl.ds(base, SE)], src_v)

        @pl.loop(0, SUBG)
        def _group(gg):
            for h in range(H):
                for j in range(G // L):
                    o = gg * G + j * L
                    tg = tgt_v[pl.ds(o, L)]
                    r = plsc.load_gather(nr_v, [tg + h * N_NODES])
                    a = al_v[pl.ds(h * SE + o, L)] + r
                    ex = jnp.exp(jnp.maximum(a, 0.2 * a))
                    ex_v[pl.ds(h * SE + o, L)] = ex
                    sv = src_v[pl.ds(o, L)]
                    idx_v[h * SUBG + gg, pl.ds(j * L, L)] = sv * H + h

        for h in range(H):
            pltpu.sync_copy(ex_v.at[pl.ds(h * SE, SE)],
                            ex_hbm.at[pl.ds(h * E_PAD + base, SE)])

        @pl.loop(0, SUBG)
        def _fire(gg):
            for h in range(H):
                pltpu.async_copy(ex_v.at[pl.ds(h * SE + gg * G, G)],
                                 dacc.at[idx_v.at[h * SUBG + gg]],
                                 ssem, add=True)

        pltpu.make_async_copy(al_hbm.at[pl.ds(0, H * SE)], ex_v, ssem).wait()

    plsc.subcore_barrier()
    pltpu.sync_copy(dacc.at[pl.ds(s * ROWS * H, ROWS * H)],
                    dparts_hbm.at[pl.ds((c * NS + s) * ROWS * H, ROWS * H)])


@functools.partial(
    pl.kernel,
    out_type=jax.ShapeDtypeStruct((NC, N_ACC, D), jnp.float32),
    mesh=_SC_MESH,
    scratch_types=[
        pltpu.VMEM((G, D), jnp.float32),
        pltpu.VMEM((G, D), jnp.float32),
        pltpu.VMEM((GROUPS, G), jnp.int32),
        pltpu.VMEM_SHARED((N_ACC, D), jnp.float32),
        pltpu.SemaphoreType.DMA,
        pltpu.SemaphoreType.DMA,
        pltpu.SemaphoreType.DMA,
        pltpu.SemaphoreType.DMA,
    ],
    compiler_params=pltpu.CompilerParams(use_tc_tiling_on_sc=False, needs_layout_passes=False),
)
def _sc_scatter(y_hbm, src_hbm, zout_hbm, oparts_hbm,
                y0, y1, srcall, oacc, l0, l1, s0, s1):
    c = lax.axis_index("c")
    s = lax.axis_index("s")
    wid = c * NS + s
    ybufs = (y0, y1)
    lsems = (l0, l1)
    ssems = (s0, s1)

    pltpu.sync_copy(zout_hbm.at[pl.ds(s * ROWS, ROWS)], oacc.at[pl.ds(s * ROWS, ROWS)])
    pltpu.sync_copy(src_hbm.at[pl.ds(wid * GROUPS, GROUPS)], srcall)
    plsc.subcore_barrier()

    tbase = wid * GROUPS * G
    pltpu.async_copy(y_hbm.at[pl.ds(tbase, G)], y0, l0)
    pltpu.async_copy(y_hbm.at[pl.ds(tbase + G, G)], y1, l1)

    @pl.loop(0, GROUPS, step=2)
    def _g(g0):
        for b in range(2):
            g = g0 + b
            bo = 1 - b

            @pl.when(g < GROUPS)
            def _():
                pltpu.make_async_copy(y_hbm.at[pl.ds(0, G)], ybufs[b], lsems[b]).wait()
                pltpu.async_copy(ybufs[b], oacc.at[srcall.at[g]], ssems[b], add=True)

                @pl.when(jnp.logical_and(g >= 1, g + 1 < GROUPS))
                def _():
                    pltpu.make_async_copy(y_hbm.at[pl.ds(0, G)], ybufs[bo], ssems[bo]).wait()
                    pltpu.async_copy(y_hbm.at[pl.ds(tbase + (g + 1) * G, G)],
                                     ybufs[bo], lsems[bo])

    pltpu.make_async_copy(y_hbm.at[pl.ds(0, G)], y0, s0).wait()
    pltpu.make_async_copy(y_hbm.at[pl.ds(0, G)], y1, s1).wait()

    plsc.subcore_barrier()
    pltpu.sync_copy(oacc.at[pl.ds(s * ROWS, ROWS)],
                    oparts_hbm.at[c, pl.ds(s * ROWS, ROWS)])


def kernel(edge_attr, edge_index, W, att_l, att_r, W_res):
    src = edge_index[1].astype(jnp.int32)
    tgt = edge_index[0].astype(jnp.int32)
    npad = E_PAD - E_TOTAL
    pad_src = (jnp.arange(npad, dtype=jnp.int32) % L) + N_NODES
    src_p = jnp.concatenate([src, pad_src])
    tgt_p = jnp.concatenate([tgt, jnp.zeros((npad,), jnp.int32)])

    filtered = edge_attr[E_RAND:]
    attl2 = att_l.reshape(1, H * C)
    attr2 = att_r.reshape(1, H * C)
    zden = jnp.zeros((N_ACC * H,), jnp.float32)
    zout = jnp.zeros((N_ACC, D), jnp.float32)

    egrid = pl.cdiv(E_TOTAL, EB)

    alpha_l = pl.pallas_call(
        _alpha_l_body,
        grid=(egrid,),
        in_specs=[
            pl.BlockSpec((EB, D), lambda i: (i, 0)),
            pl.BlockSpec((H * C, D), lambda i: (0, 0)),
            pl.BlockSpec((1, H * C), lambda i: (0, 0)),
        ],
        out_specs=pl.BlockSpec((H, EB), lambda i: (0, i)),
        out_shape=jax.ShapeDtypeStruct((H, E_PAD), jnp.float32),
    )(edge_attr, W, attl2)

    node_r, res = pl.pallas_call(
        _node_body,
        grid=(pl.cdiv(N_NODES, 2048),),
        in_specs=[
            pl.BlockSpec((2048, D), lambda i: (i, 0)),
            pl.BlockSpec((H * C, D), lambda i: (0, 0)),
            pl.BlockSpec((1, H * C), lambda i: (0, 0)),
            pl.BlockSpec((H * C, D), lambda i: (0, 0)),
        ],
        out_specs=[
            pl.BlockSpec((H, 2048), lambda i: (0, i)),
            pl.BlockSpec((2048, H * C), lambda i: (i, 0)),
        ],
        out_shape=[
            jax.ShapeDtypeStruct((H, N_NODES), jnp.float32),
            jax.ShapeDtypeStruct((N_NODES, H * C), jnp.float32),
        ],
    )(filtered, W, attr2, W_res)

    ex_flat, dparts_flat = _sc_softmax(alpha_l.reshape(-1), node_r.reshape(-1),
                                       tgt_p, src_p, zden)
    ex = ex_flat.reshape(H, E_PAD)
    dparts = dparts_flat.reshape(NC, N_ACC, H)

    y = pl.pallas_call(
        _y_body,
        grid=(egrid,),
        in_specs=[
            pl.BlockSpec((EB, D), lambda i: (i, 0)),
            pl.BlockSpec((H, EB), lambda i: (0, i)),
            pl.BlockSpec((H * C, D), lambda i: (0, 0)),
        ],
        out_specs=pl.BlockSpec((EB, D), lambda i: (i, 0)),
        out_shape=jax.ShapeDtypeStruct((E_PAD, D), jnp.float32),
    )(edge_attr, ex, W)

    oparts = _sc_scatter(y, src_p.reshape(NC * NS * GROUPS, G), zout)

    out = pl.pallas_call(
        _final_body,
        grid=(N_NODES // NB,),
        in_specs=[
            pl.BlockSpec((NC, NB, D), lambda i: (0, i, 0)),
            pl.BlockSpec((NC, NB, H), lambda i: (0, i, 0)),
            pl.BlockSpec((NB, H * C), lambda i: (i, 0)),
        ],
        out_specs=pl.BlockSpec((NB, H * C), lambda i: (i, 0)),
        out_shape=jax.ShapeDtypeStruct((N_NODES, H * C), jnp.float32),
    )(oparts, dparts, res)

    return out


# speedup vs baseline: 83.7086x; 1.0097x over previous
import functools

import jax
import jax.numpy as jnp
from jax import lax
from jax.experimental import pallas as pl
from jax.experimental.pallas import tpu as pltpu
from jax.experimental.pallas import tpu_sc as plsc

N_NODES = 10000
E_RAND = 320000
E_TOTAL = E_RAND + N_NODES
D = 128
H = 4
C = 32

NC = 2
NS = 16
L = 16

G = 128
GROUPS = 81
E_PAD = NC * NS * GROUPS * G
N_ACC = 10112
ROWS = N_ACC // NS

EB = 4096
NB = 1000


def _hsel():
    hh = lax.broadcasted_iota(jnp.int32, (H, H * C), 0)
    jj = lax.broadcasted_iota(jnp.int32, (H, H * C), 1)
    return (jj // C == hh).astype(jnp.float32)


def _alpha_l_body(ea_ref, w_ref, attl_ref, out_ref):
    wv = w_ref[:, :] * attl_ref[0, :][:, None]
    vlt = lax.dot_general(_hsel(), wv, (((1,), (0,)), ((), ())),
                          preferred_element_type=jnp.float32)
    out_ref[:, :] = lax.dot_general(vlt, ea_ref[:, :], (((1,), (1,)), ((), ())),
                                    preferred_element_type=jnp.float32)


def _node_body(f_ref, w_ref, attr_ref, wres_ref, nr_ref, res_ref):
    f = f_ref[:, :]
    xi = lax.dot_general(f, w_ref[:, :], (((1,), (1,)), ((), ())),
                         preferred_element_type=jnp.float32)
    xia = xi * attr_ref[0, :][None, :]
    nr_ref[:, :] = lax.dot_general(_hsel(), xia, (((1,), (1,)), ((), ())),
                                   preferred_element_type=jnp.float32)
    res_ref[:, :] = lax.dot_general(f, wres_ref[:, :], (((1,), (1,)), ((), ())),
                                    preferred_element_type=jnp.float32)


def _y_body(ea_ref, ex_ref, w_ref, y_ref):
    xj = lax.dot_general(ea_ref[:, :], w_ref[:, :], (((1,), (1,)), ((), ())),
                         preferred_element_type=jnp.float32)
    scale = lax.dot_general(ex_ref[:, :], _hsel(), (((0,), (0,)), ((), ())),
                            preferred_element_type=jnp.float32)
    y_ref[:, :] = xj * scale


def _final_body(sa_ref, sb_ref, dp_ref, res_ref, out_ref):
    s = sa_ref[0] + sa_ref[1] + sb_ref[0] + sb_ref[1]
    d = dp_ref[0] + dp_ref[1]
    dexp = lax.dot_general(d, _hsel(), (((1,), (0,)), ((), ())),
                           preferred_element_type=jnp.float32) + 1e-16
    pre = s / dexp
    act = jnp.where(pre > 0, pre, jnp.exp(jnp.minimum(pre, 0.0)) - 1.0)
    out_ref[:, :] = act + res_ref[:, :]


_SC_MESH = plsc.VectorSubcoreMesh(core_axis_name="c", subcore_axis_name="s")


SUBG = 27
NSUB = GROUPS // SUBG
SE = SUBG * G


@functools.partial(
    pl.kernel,
    out_type=[
        jax.ShapeDtypeStruct((H * E_PAD,), jnp.float32),
        jax.ShapeDtypeStruct((NC * N_ACC * H,), jnp.float32),
    ],
    mesh=_SC_MESH,
    scratch_types=[
        pltpu.VMEM((H * N_NODES,), jnp.float32),
        pltpu.VMEM((H * SE,), jnp.float32),
        pltpu.VMEM((H * SE,), jnp.float32),
        pltpu.VMEM((SE,), jnp.int32),
        pltpu.VMEM((SE,), jnp.int32),
        pltpu.VMEM((H * SUBG, G), jnp.int32),
        pltpu.VMEM_SHARED((N_ACC * H,), jnp.float32),
        pltpu.SemaphoreType.DMA,
    ],
    compiler_params=pltpu.CompilerParams(use_tc_tiling_on_sc=False, needs_layout_passes=False),
)
def _sc_softmax(al_hbm, nr_hbm, tgt_hbm, src_hbm, zden_hbm,
                ex_hbm, dparts_hbm,
                nr_v, al_v, ex_v, tgt_v, src_v, idx_v, dacc, ssem):
    c = lax.axis_index("c")
    s = lax.axis_index("s")
    wid = c * NS + s

    pltpu.sync_copy(zden_hbm.at[pl.ds(s * ROWS * H, ROWS * H)],
                    dacc.at[pl.ds(s * ROWS * H, ROWS * H)])
    pltpu.sync_copy(nr_hbm, nr_v)
    plsc.subcore_barrier()

    @pl.loop(0, NSUB)
    def _sub(sub):
        base = wid * GROUPS * G + sub * SE
        for h in range(H):
            pltpu.sync_copy(al_hbm.at[pl.ds(h * E_PAD + base, SE)],
                            al_v.at[pl.ds(h * SE, SE)])
        pltpu.sync_copy(tgt_hbm.at[pl.ds(base, SE)], tgt_v)
        pltpu.sync_copy(src_hbm.at[pl.ds(base, SE)], src_v)

        @pl.loop(0, SUBG)
        def _group(gg):
            for h in range(H):
                for j in range(G // L):
                    o = gg * G + j * L
                    tg = tgt_v[pl.ds(o, L)]
                    r = plsc.load_gather(nr_v, [tg + h * N_NODES])
                    a = al_v[pl.ds(h * SE + o, L)] + r
                    ex = jnp.exp(jnp.maximum(a, 0.2 * a))
                    ex_v[pl.ds(h * SE + o, L)] = ex
                    sv = src_v[pl.ds(o, L)]
                    idx_v[h * SUBG + gg, pl.ds(j * L, L)] = sv * H + h

        for h in range(H):
            pltpu.sync_copy(ex_v.at[pl.ds(h * SE, SE)],
                            ex_hbm.at[pl.ds(h * E_PAD + base, SE)])

        @pl.loop(0, SUBG)
        def _fire(gg):
            for h in range(H):
                pltpu.async_copy(ex_v.at[pl.ds(h * SE + gg * G, G)],
                                 dacc.at[idx_v.at[h * SUBG + gg]],
                                 ssem, add=True)

        pltpu.make_async_copy(al_hbm.at[pl.ds(0, H * SE)], ex_v, ssem).wait()

    plsc.subcore_barrier()
    pltpu.sync_copy(dacc.at[pl.ds(s * ROWS * H, ROWS * H)],
                    dparts_hbm.at[pl.ds((c * NS + s) * ROWS * H, ROWS * H)])


def _make_sc_scatter(n_groups, gbase):
    @functools.partial(
        pl.kernel,
        out_type=jax.ShapeDtypeStruct((NC, N_ACC, D), jnp.float32),
        mesh=_SC_MESH,
        scratch_types=[
            pltpu.VMEM((G, D), jnp.float32),
            pltpu.VMEM((G, D), jnp.float32),
            pltpu.VMEM((n_groups, G), jnp.int32),
            pltpu.VMEM_SHARED((N_ACC, D), jnp.float32),
            pltpu.SemaphoreType.DMA,
            pltpu.SemaphoreType.DMA,
            pltpu.SemaphoreType.DMA,
            pltpu.SemaphoreType.DMA,
        ],
        compiler_params=pltpu.CompilerParams(use_tc_tiling_on_sc=False, needs_layout_passes=False),
    )
    def _sc_scatter(y_hbm, src_hbm, zout_hbm, oparts_hbm,
                    y0, y1, srcall, oacc, l0, l1, s0, s1):
        c = lax.axis_index("c")
        s = lax.axis_index("s")
        wid = c * NS + s
        ybufs = (y0, y1)
        lsems = (l0, l1)
        ssems = (s0, s1)

        pltpu.sync_copy(zout_hbm.at[pl.ds(s * ROWS, ROWS)], oacc.at[pl.ds(s * ROWS, ROWS)])
        pltpu.sync_copy(src_hbm.at[pl.ds(gbase + wid * n_groups, n_groups)], srcall)
        plsc.subcore_barrier()

        tbase = wid * n_groups * G
        pltpu.async_copy(y_hbm.at[pl.ds(tbase, G)], y0, l0)
        pltpu.async_copy(y_hbm.at[pl.ds(tbase + G, G)], y1, l1)

        @pl.loop(0, n_groups, step=2)
        def _g(g0):
            for b in range(2):
                g = g0 + b
                bo = 1 - b

                @pl.when(g < n_groups)
                def _():
                    pltpu.make_async_copy(y_hbm.at[pl.ds(0, G)], ybufs[b], lsems[b]).wait()
                    pltpu.async_copy(ybufs[b], oacc.at[srcall.at[g]], ssems[b], add=True)

                    @pl.when(jnp.logical_and(g >= 1, g + 1 < n_groups))
                    def _():
                        pltpu.make_async_copy(y_hbm.at[pl.ds(0, G)], ybufs[bo], ssems[bo]).wait()
                        pltpu.async_copy(y_hbm.at[pl.ds(tbase + (g + 1) * G, G)],
                                         ybufs[bo], lsems[bo])

        pltpu.make_async_copy(y_hbm.at[pl.ds(0, G)], y0, s0).wait()
        pltpu.make_async_copy(y_hbm.at[pl.ds(0, G)], y1, s1).wait()

        plsc.subcore_barrier()
        pltpu.sync_copy(oacc.at[pl.ds(s * ROWS, ROWS)],
                        oparts_hbm.at[c, pl.ds(s * ROWS, ROWS)])

    return _sc_scatter


GROUPS_A = 54
GROUPS_B = GROUPS - GROUPS_A
E_SPLIT = NC * NS * GROUPS_A * G
_sc_scatter_a = _make_sc_scatter(GROUPS_A, 0)
_sc_scatter_b = _make_sc_scatter(GROUPS_B, NC * NS * GROUPS_A)


def kernel(edge_attr, edge_index, W, att_l, att_r, W_res):
    src = edge_index[1].astype(jnp.int32)
    tgt = edge_index[0].astype(jnp.int32)
    npad = E_PAD - E_TOTAL
    pad_src = (jnp.arange(npad, dtype=jnp.int32) % L) + N_NODES
    src_p = jnp.concatenate([src, pad_src])
    tgt_p = jnp.concatenate([tgt, jnp.zeros((npad,), jnp.int32)])

    filtered = edge_attr[E_RAND:]
    attl2 = att_l.reshape(1, H * C)
    attr2 = att_r.reshape(1, H * C)
    zden = jnp.zeros((N_ACC * H,), jnp.float32)
    zout = jnp.zeros((N_ACC, D), jnp.float32)

    egrid = pl.cdiv(E_TOTAL, EB)

    alpha_l = pl.pallas_call(
        _alpha_l_body,
        grid=(egrid,),
        in_specs=[
            pl.BlockSpec((EB, D), lambda i: (i, 0)),
            pl.BlockSpec((H * C, D), lambda i: (0, 0)),
            pl.BlockSpec((1, H * C), lambda i: (0, 0)),
        ],
        out_specs=pl.BlockSpec((H, EB), lambda i: (0, i)),
        out_shape=jax.ShapeDtypeStruct((H, E_PAD), jnp.float32),
    )(edge_attr, W, attl2)

    node_r, res = pl.pallas_call(
        _node_body,
        grid=(pl.cdiv(N_NODES, 2048),),
        in_specs=[
            pl.BlockSpec((2048, D), lambda i: (i, 0)),
            pl.BlockSpec((H * C, D), lambda i: (0, 0)),
            pl.BlockSpec((1, H * C), lambda i: (0, 0)),
            pl.BlockSpec((H * C, D), lambda i: (0, 0)),
        ],
        out_specs=[
            pl.BlockSpec((H, 2048), lambda i: (0, i)),
            pl.BlockSpec((2048, H * C), lambda i: (i, 0)),
        ],
        out_shape=[
            jax.ShapeDtypeStruct((H, N_NODES), jnp.float32),
            jax.ShapeDtypeStruct((N_NODES, H * C), jnp.float32),
        ],
    )(filtered, W, attr2, W_res)

    ex_flat, dparts_flat = _sc_softmax(alpha_l.reshape(-1), node_r.reshape(-1),
                                       tgt_p, src_p, zden)
    ex = ex_flat.reshape(H, E_PAD)
    dparts = dparts_flat.reshape(NC, N_ACC, H)

    ga = GROUPS_A * NC * NS // (EB // G)
    gb = GROUPS_B * NC * NS // (EB // G)
    src2d = src_p.reshape(NC * NS * GROUPS, G)

    y_a = pl.pallas_call(
        _y_body,
        grid=(ga,),
        in_specs=[
            pl.BlockSpec((EB, D), lambda i: (i, 0)),
            pl.BlockSpec((H, EB), lambda i: (0, i)),
            pl.BlockSpec((H * C, D), lambda i: (0, 0)),
        ],
        out_specs=pl.BlockSpec((EB, D), lambda i: (i, 0)),
        out_shape=jax.ShapeDtypeStruct((E_SPLIT, D), jnp.float32),
    )(edge_attr, ex, W)

    y_b = pl.pallas_call(
        _y_body,
        grid=(gb,),
        in_specs=[
            pl.BlockSpec((EB, D), lambda i: (i + ga, 0)),
            pl.BlockSpec((H, EB), lambda i: (0, i + ga)),
            pl.BlockSpec((H * C, D), lambda i: (0, 0)),
        ],
        out_specs=pl.BlockSpec((EB, D), lambda i: (i, 0)),
        out_shape=jax.ShapeDtypeStruct((E_PAD - E_SPLIT, D), jnp.float32),
    )(edge_attr, ex, W)

    oparts_a = _sc_scatter_a(y_a, src2d, zout)
    oparts_b = _sc_scatter_b(y_b, src2d, zout)

    out = pl.pallas_call(
        _final_body,
        grid=(N_NODES // NB,),
        in_specs=[
            pl.BlockSpec((NC, NB, D), lambda i: (0, i, 0)),
            pl.BlockSpec((NC, NB, D), lambda i: (0, i, 0)),
            pl.BlockSpec((NC, NB, H), lambda i: (0, i, 0)),
            pl.BlockSpec((NB, H * C), lambda i: (i, 0)),
        ],
        out_specs=pl.BlockSpec((NB, H * C), lambda i: (i, 0)),
        out_shape=jax.ShapeDtypeStruct((N_NODES, H * C), jnp.float32),
    )(oparts_a, oparts_b, dparts, res)

    return out
